# Initial kernel scaffold; baseline (speedup 1.0000x reference)
import jax
import jax.numpy as jnp
from jax.experimental import pallas as pl


def kernel(input, seq_lens, W_x, W_h, b, init_state, batch_size, depth, output_size):
    raise NotImplementedError("write your pallas kernel here")


# speedup vs baseline: 12.2439x; 12.2439x over previous
import jax
import jax.numpy as jnp
from jax.experimental import pallas as pl
from jax.experimental.pallas import tpu as pltpu


def _rnn_body(seq_ref, x_ref, wx_ref, wh_ref, b_ref, init_ref, *refs, chunk, depth):
    out_refs = refs[:depth]
    carry_ref = refs[depth]
    c = pl.program_id(0)

    @pl.when(c == 0)
    def _():
        carry_ref[...] = jnp.broadcast_to(init_ref[...], carry_ref.shape)

    seq = seq_ref[...]

    def step(i, _):
        gt = c * chunk + i
        mask = seq > gt
        layer_in = x_ref[:, i, :]
        for k in range(depth):
            h = jnp.tanh(
                jnp.dot(layer_in, wx_ref[k], preferred_element_type=jnp.float32)
                + jnp.dot(carry_ref[k], wh_ref[k], preferred_element_type=jnp.float32)
                + b_ref[k]
            )
            carry_ref[k] = h
            out_refs[k][:, i, :] = jnp.where(mask, h, 0.0)
            layer_in = h
        return 0

    jax.lax.fori_loop(0, chunk, step, 0, unroll=True)


def kernel(input, seq_lens, W_x, W_h, b, init_state, batch_size, depth, output_size):
    B, S, H = input.shape
    DEPTH = W_x.shape[0]
    CHUNK = 128
    grid = (S // CHUNK,)

    seq2d = seq_lens.reshape(B, 1)
    b3d = b.reshape(DEPTH, 1, H)

    outs = pl.pallas_call(
        lambda *refs: _rnn_body(*refs, chunk=CHUNK, depth=DEPTH),
        grid=grid,
        in_specs=[
            pl.BlockSpec((B, 1), lambda c: (0, 0)),
            pl.BlockSpec((B, CHUNK, H), lambda c: (0, c, 0)),
            pl.BlockSpec((DEPTH, H, H), lambda c: (0, 0, 0)),
            pl.BlockSpec((DEPTH, H, H), lambda c: (0, 0, 0)),
            pl.BlockSpec((DEPTH, 1, H), lambda c: (0, 0, 0)),
            pl.BlockSpec((1, H), lambda c: (0, 0)),
        ],
        out_specs=tuple(
            pl.BlockSpec((B, CHUNK, H), lambda c: (0, c, 0)) for _ in range(DEPTH)
        ),
        out_shape=tuple(
            jax.ShapeDtypeStruct((B, S, H), jnp.float32) for _ in range(DEPTH)
        ),
        scratch_shapes=[pltpu.VMEM((DEPTH, B, H), jnp.float32)],
    )(seq2d, input, W_x, W_h, b3d, init_state)

    return jnp.stack(outs, axis=2)


# speedup vs baseline: 12.4899x; 1.0201x over previous
import jax
import jax.numpy as jnp
from jax.experimental import pallas as pl
from jax.experimental.pallas import tpu as pltpu


def _rnn_body(seq_ref, x_ref, wx_ref, wh_ref, b_ref, init_ref, *refs,
              chunk, depth):
    out_refs = refs[:depth]
    carry_ref = refs[depth]
    xp_ref = refs[depth + 1]
    c = pl.program_id(0)

    @pl.when(c == 0)
    def _():
        carry_ref[...] = jnp.broadcast_to(init_ref[...], carry_ref.shape)

    xp_ref[...] = jax.lax.dot_general(
        x_ref[...], wx_ref[0],
        (((2,), (0,)), ((), ())),
        preferred_element_type=jnp.float32,
    ) + b_ref[0][None]

    def step(i, _):
        h = jnp.tanh(
            xp_ref[:, i, :]
            + jnp.dot(carry_ref[0], wh_ref[0], preferred_element_type=jnp.float32)
        )
        carry_ref[0] = h
        out_refs[0][:, i, :] = h
        for k in range(1, depth):
            h = jnp.tanh(
                jnp.dot(h, wx_ref[k], preferred_element_type=jnp.float32)
                + jnp.dot(carry_ref[k], wh_ref[k], preferred_element_type=jnp.float32)
                + b_ref[k]
            )
            carry_ref[k] = h
            out_refs[k][:, i, :] = h
        return 0

    jax.lax.fori_loop(0, chunk, step, 0, unroll=True)

    t_ids = jax.lax.broadcasted_iota(jnp.int32, (1, chunk, 1), 1) + c * chunk
    mask = t_ids < seq_ref[...][:, None, :]
    for k in range(depth):
        out_refs[k][...] = jnp.where(mask, out_refs[k][...], 0.0)


def kernel(input, seq_lens, W_x, W_h, b, init_state, batch_size, depth, output_size):
    B, S, H = input.shape
    DEPTH = W_x.shape[0]
    CHUNK = 128
    grid = (S // CHUNK,)

    seq2d = seq_lens.reshape(B, 1)
    b3d = b.reshape(DEPTH, 1, H)

    outs = pl.pallas_call(
        lambda *refs: _rnn_body(*refs, chunk=CHUNK, depth=DEPTH),
        grid=grid,
        in_specs=[
            pl.BlockSpec((B, 1), lambda c: (0, 0)),
            pl.BlockSpec((B, CHUNK, H), lambda c: (0, c, 0)),
            pl.BlockSpec((DEPTH, H, H), lambda c: (0, 0, 0)),
            pl.BlockSpec((DEPTH, H, H), lambda c: (0, 0, 0)),
            pl.BlockSpec((DEPTH, 1, H), lambda c: (0, 0, 0)),
            pl.BlockSpec((1, H), lambda c: (0, 0)),
        ],
        out_specs=tuple(
            pl.BlockSpec((B, CHUNK, H), lambda c: (0, c, 0)) for _ in range(DEPTH)
        ),
        out_shape=tuple(
            jax.ShapeDtypeStruct((B, S, H), jnp.float32) for _ in range(DEPTH)
        ),
        scratch_shapes=[
            pltpu.VMEM((DEPTH, B, H), jnp.float32),
            pltpu.VMEM((B, CHUNK, H), jnp.float32),
        ],
    )(seq2d, input, W_x, W_h, b3d, init_state)

    return jnp.stack(outs, axis=2)
